# Initial kernel scaffold; baseline (speedup 1.0000x reference)
import jax
import jax.numpy as jnp
from jax.experimental import pallas as pl


def kernel(x, edge_index, edge_attr, u, batch, W1, b1, W2, b2):
    raise NotImplementedError("write your pallas kernel here")


# speedup vs baseline: 22.3243x; 22.3243x over previous
import functools

import jax
import jax.numpy as jnp
from jax import lax
from jax.experimental import pallas as pl
from jax.experimental.pallas import tpu as pltpu
from jax.experimental.pallas import tpu_sc as plsc

_L = 16


def _make_edge_kernel(E, F, N, B, nc, ns):
    NW = nc * ns
    C = 1280
    SUB = 128
    NSUB = C // SUB
    assert E % C == 0
    n_chunks = E // C
    base_k = n_chunks // NW
    rem = n_chunks % NW

    mesh = plsc.VectorSubcoreMesh(core_axis_name="c", subcore_axis_name="s")

    @functools.partial(
        pl.kernel,
        out_type=(
            jax.ShapeDtypeStruct((nc, B, F), jnp.float32),
            jax.ShapeDtypeStruct((NW, _L, B), jnp.float32),
        ),
        mesh=mesh,
        compiler_params=pltpu.CompilerParams(needs_layout_passes=False,
                                             use_tc_tiling_on_sc=False),
        scratch_types=[
            pltpu.VMEM((N,), jnp.int32),
            pltpu.VMEM((C,), jnp.int32),
            pltpu.VMEM((C, F), jnp.float32),
            pltpu.VMEM((NSUB, SUB), jnp.int32),
            pltpu.VMEM_SHARED((B, F), jnp.float32),
            pltpu.VMEM((_L, B), jnp.float32),
        ],
    )
    def edge_kernel(row_hbm, attr_hbm, batch_hbm, sum_hbm, cnt_hbm,
                    batch_v, row_v, attr_v, seg_v, acc_s, cnt_v):
        cid = lax.axis_index("c")
        sid = lax.axis_index("s")
        wid = sid * nc + cid

        zf = jnp.zeros((_L,), jnp.float32)

        def zattr(i, c):
            attr_v[i, :] = zf
            return c

        lax.fori_loop(0, B, zattr, 0)

        @pl.when(sid == 0)
        def _():
            pltpu.sync_copy(attr_v.at[pl.ds(0, B)], acc_s)

        def zcnt(i, c):
            cnt_v[i // (B // _L), pl.ds((i % (B // _L)) * _L, _L)] = zf
            return c

        lax.fori_loop(0, _L * (B // _L), zcnt, 0)

        pltpu.sync_copy(batch_hbm, batch_v)
        plsc.subcore_barrier()

        lanes = lax.iota(jnp.int32, _L)
        ones = jnp.ones((_L,), jnp.float32)
        nk = jnp.where(wid < rem, base_k + 1, base_k)

        def chunk_body(k, c):
            start = (wid + k * NW) * C
            pltpu.sync_copy(row_hbm.at[pl.ds(start, C)], row_v)
            pltpu.sync_copy(attr_hbm.at[pl.ds(start, C)], attr_v)
            for s in range(NSUB):
                def seg_body(j, cc):
                    rv = row_v[pl.ds(s * SUB + j * _L, _L)]
                    seg = plsc.load_gather(batch_v, [rv])
                    seg_v[s, pl.ds(j * _L, _L)] = seg
                    plsc.addupdate_scatter(cnt_v, [lanes, seg], ones)
                    return cc

                lax.fori_loop(0, SUB // _L, seg_body, 0)
                pltpu.sync_copy(attr_v.at[pl.ds(s * SUB, SUB)],
                                acc_s.at[seg_v.at[s]], add=True)
            return c

        lax.fori_loop(0, nk, chunk_body, 0)
        plsc.subcore_barrier()

        @pl.when(sid == 0)
        def _():
            pltpu.sync_copy(acc_s, sum_hbm.at[cid])

        pltpu.sync_copy(cnt_v, cnt_hbm.at[wid])

    return edge_kernel


def _make_node_kernel(N, D, B, NB):
    G = N // NB
    assert N % NB == 0

    def body(batch_ref, x_ref, sum_ref, cnt_ref):
        @pl.when(pl.program_id(0) == 0)
        def _():
            sum_ref[...] = jnp.zeros_like(sum_ref)
            cnt_ref[...] = jnp.zeros_like(cnt_ref)

        b = batch_ref[0, 0, :]
        onehot = (b[None, :] == lax.broadcasted_iota(jnp.int32, (B, NB), 0)
                  ).astype(jnp.float32)
        sum_ref[...] += jnp.dot(onehot, x_ref[...],
                                preferred_element_type=jnp.float32)
        cnt_ref[...] += jnp.broadcast_to(
            jnp.sum(onehot, axis=1, keepdims=True), cnt_ref.shape)

    return pl.pallas_call(
        body,
        grid=(G,),
        in_specs=[
            pl.BlockSpec((1, 1, NB), lambda i: (i, 0, 0)),
            pl.BlockSpec((NB, D), lambda i: (i, 0)),
        ],
        out_specs=[
            pl.BlockSpec((B, D), lambda i: (0, 0)),
            pl.BlockSpec((B, D), lambda i: (0, 0)),
        ],
        out_shape=[
            jax.ShapeDtypeStruct((B, D), jnp.float32),
            jax.ShapeDtypeStruct((B, D), jnp.float32),
        ],
    )


def _make_mlp_kernel(B, D, F, U, H, O):
    def body(u_ref, nsum_ref, ncnt_ref, esum_ref, ecnt_ref,
             w1_ref, b1_ref, w2_ref, b2_ref, out_ref):
        esum = jnp.sum(esum_ref[...], axis=0)
        ecnt = jnp.sum(ecnt_ref[...], axis=(0, 1))
        emean = esum / jnp.maximum(ecnt, 1.0)[:, None]
        nmean = nsum_ref[...] / jnp.maximum(ncnt_ref[...], 1.0)
        h = (jnp.dot(u_ref[...], w1_ref[0:U, :],
                     preferred_element_type=jnp.float32)
             + jnp.dot(nmean, w1_ref[U:U + D, :],
                       preferred_element_type=jnp.float32)
             + jnp.dot(emean, w1_ref[U + D:U + D + F, :],
                       preferred_element_type=jnp.float32)
             + b1_ref[...])
        h = jnp.maximum(h, 0.0)
        out_ref[...] = jnp.dot(h, w2_ref[...],
                               preferred_element_type=jnp.float32) + b2_ref[...]

    return pl.pallas_call(
        body,
        out_shape=jax.ShapeDtypeStruct((B, O), jnp.float32),
    )


def kernel(x, edge_index, edge_attr, u, batch, W1, b1, W2, b2):
    N, D = x.shape
    E, F = edge_attr.shape
    B, U = u.shape
    H = W1.shape[1]
    O = W2.shape[1]
    row = edge_index[0]

    info = plsc.get_sparse_core_info()
    esum, ecnt = _make_edge_kernel(E, F, N, B, info.num_cores,
                                   info.num_subcores)(row, edge_attr, batch)

    NB = 2000
    nsum, ncnt = _make_node_kernel(N, D, B, NB)(
        batch.reshape(N // NB, 1, NB), x)

    return _make_mlp_kernel(B, D, F, U, H, O)(
        u, nsum, ncnt, esum, ecnt, W1,
        b1.reshape(1, H), W2, b2.reshape(1, O))


# speedup vs baseline: 25.3305x; 1.1347x over previous
import functools

import jax
import jax.numpy as jnp
from jax import lax
from jax.experimental import pallas as pl
from jax.experimental.pallas import tpu as pltpu
from jax.experimental.pallas import tpu_sc as plsc

_L = 16


def _make_edge_kernel(E, F, N, B, nc, ns):
    NW = nc * ns
    PW = E // NW
    C = 2000
    assert E % NW == 0 and PW % C == 0 and C % _L == 0
    KW = PW // C

    mesh = plsc.VectorSubcoreMesh(core_axis_name="c", subcore_axis_name="s")

    @functools.partial(
        pl.kernel,
        out_type=(
            jax.ShapeDtypeStruct((nc, B, F), jnp.float32),
            jax.ShapeDtypeStruct((NW, _L, B), jnp.float32),
        ),
        mesh=mesh,
        compiler_params=pltpu.CompilerParams(needs_layout_passes=False,
                                             use_tc_tiling_on_sc=False),
        scratch_types=[
            pltpu.VMEM((N,), jnp.int32),
            [pltpu.VMEM((C,), jnp.int32)] * 2,
            [pltpu.VMEM((C, F), jnp.float32)] * 2,
            [pltpu.VMEM((C,), jnp.int32)] * 2,
            pltpu.VMEM_SHARED((B, F), jnp.float32),
            pltpu.VMEM((_L, B), jnp.float32),
            [pltpu.SemaphoreType.DMA] * 2,
            [pltpu.SemaphoreType.DMA] * 2,
        ],
    )
    def edge_kernel(ei_hbm, attr_hbm, batch_hbm, sum_hbm, cnt_hbm,
                    batch_v, row_v, attr_v, seg_v, acc_s, cnt_v,
                    dsem, ssem):
        cid = lax.axis_index("c")
        sid = lax.axis_index("s")
        wid = sid * nc + cid

        zf = jnp.zeros((_L,), jnp.float32)

        def zattr(i, c):
            attr_v[0][i, :] = zf
            return c

        lax.fori_loop(0, B, zattr, 0)

        @pl.when(sid == 0)
        def _():
            pltpu.sync_copy(attr_v[0].at[pl.ds(0, B)], acc_s)

        def zcnt(i, c):
            cnt_v[i // (B // _L), pl.ds((i % (B // _L)) * _L, _L)] = zf
            return c

        lax.fori_loop(0, _L * (B // _L), zcnt, 0)

        pltpu.sync_copy(batch_hbm, batch_v)
        plsc.subcore_barrier()

        lanes = lax.iota(jnp.int32, _L)
        ones = jnp.ones((_L,), jnp.float32)
        base = wid * PW

        def start_dma(k):
            b = k % 2
            s = base + k * C
            return (pltpu.async_copy(ei_hbm.at[0, pl.ds(s, C)], row_v[b],
                                     dsem[b]),
                    pltpu.async_copy(attr_hbm.at[pl.ds(s, C)], attr_v[b],
                                     dsem[b]))

        dma = {0: start_dma(0)}
        streams = {}
        for k in range(KW):
            b = k % 2
            nb = (k + 1) % 2
            if k >= 1:
                streams.pop(k - 1).wait()
            if k + 1 < KW:
                dma[k + 1] = start_dma(k + 1)
            d1, d2 = dma.pop(k)
            d1.wait()
            d2.wait()

            def seg_body(j, c, b=b):
                rv = row_v[b][pl.ds(j * _L, _L)]
                seg = plsc.load_gather(batch_v, [rv])
                seg_v[b][pl.ds(j * _L, _L)] = seg
                plsc.addupdate_scatter(cnt_v, [lanes, seg], ones)
                return c

            lax.fori_loop(0, C // _L, seg_body, 0)
            streams[k] = pltpu.async_copy(attr_v[b], acc_s.at[seg_v[b]],
                                          ssem[b], add=True)
        streams.pop(KW - 1).wait()
        plsc.subcore_barrier()

        @pl.when(sid == 0)
        def _():
            pltpu.sync_copy(acc_s, sum_hbm.at[cid])

        pltpu.sync_copy(cnt_v, cnt_hbm.at[wid])

    return edge_kernel


def _make_node_kernel(N, D, B, NB):
    G = N // NB
    assert N % NB == 0

    def body(batch_ref, x_ref, sum_ref, cnt_ref):
        @pl.when(pl.program_id(0) == 0)
        def _():
            sum_ref[...] = jnp.zeros_like(sum_ref)
            cnt_ref[...] = jnp.zeros_like(cnt_ref)

        b = batch_ref[0, 0, :]
        onehot = (b[None, :] == lax.broadcasted_iota(jnp.int32, (B, NB), 0)
                  ).astype(jnp.float32)
        sum_ref[...] += jnp.dot(onehot, x_ref[...],
                                preferred_element_type=jnp.float32)
        cnt_ref[...] += jnp.broadcast_to(
            jnp.sum(onehot, axis=1, keepdims=True), cnt_ref.shape)

    return pl.pallas_call(
        body,
        grid=(G,),
        in_specs=[
            pl.BlockSpec((1, 1, NB), lambda i: (i, 0, 0)),
            pl.BlockSpec((NB, D), lambda i: (i, 0)),
        ],
        out_specs=[
            pl.BlockSpec((B, D), lambda i: (0, 0)),
            pl.BlockSpec((B, D), lambda i: (0, 0)),
        ],
        out_shape=[
            jax.ShapeDtypeStruct((B, D), jnp.float32),
            jax.ShapeDtypeStruct((B, D), jnp.float32),
        ],
    )


def _make_mlp_kernel(B, D, F, U, H, O):
    def body(u_ref, nsum_ref, ncnt_ref, esum_ref, ecnt_ref,
             w1_ref, b1_ref, w2_ref, b2_ref, out_ref):
        esum = jnp.sum(esum_ref[...], axis=0)
        ecnt = jnp.sum(ecnt_ref[...], axis=(0, 1))
        emean = esum / jnp.maximum(ecnt, 1.0)[:, None]
        nmean = nsum_ref[...] / jnp.maximum(ncnt_ref[...], 1.0)
        h = (jnp.dot(u_ref[...], w1_ref[0:U, :],
                     preferred_element_type=jnp.float32)
             + jnp.dot(nmean, w1_ref[U:U + D, :],
                       preferred_element_type=jnp.float32)
             + jnp.dot(emean, w1_ref[U + D:U + D + F, :],
                       preferred_element_type=jnp.float32)
             + b1_ref[...])
        h = jnp.maximum(h, 0.0)
        out_ref[...] = jnp.dot(h, w2_ref[...],
                               preferred_element_type=jnp.float32) + b2_ref[...]

    return pl.pallas_call(
        body,
        out_shape=jax.ShapeDtypeStruct((B, O), jnp.float32),
    )


def kernel(x, edge_index, edge_attr, u, batch, W1, b1, W2, b2):
    N, D = x.shape
    E, F = edge_attr.shape
    B, U = u.shape
    H = W1.shape[1]
    O = W2.shape[1]
    info = plsc.get_sparse_core_info()
    esum, ecnt = _make_edge_kernel(E, F, N, B, info.num_cores,
                                   info.num_subcores)(edge_index, edge_attr,
                                                      batch)

    NB = 2000
    nsum, ncnt = _make_node_kernel(N, D, B, NB)(
        batch.reshape(N // NB, 1, NB), x)

    return _make_mlp_kernel(B, D, F, U, H, O)(
        u, nsum, ncnt, esum, ecnt, W1,
        b1.reshape(1, H), W2, b2.reshape(1, O))


# speedup vs baseline: 25.7345x; 1.0159x over previous
import functools

import jax
import jax.numpy as jnp
from jax import lax
from jax.experimental import pallas as pl
from jax.experimental.pallas import tpu as pltpu
from jax.experimental.pallas import tpu_sc as plsc

_L = 16


def _make_edge_kernel(E, F, N, B, nc, ns):
    NW = nc * ns
    NP = F // 8
    NTILES = E // 128
    MAIN = NTILES // NW
    TAIL = NTILES % NW
    NT = 5
    C = NT * 128
    assert E % 128 == 0 and MAIN % NT == 0 and F % 8 == 0
    KW = MAIN // NT
    assert KW % 2 == 0

    mesh = plsc.VectorSubcoreMesh(core_axis_name="c", subcore_axis_name="s")

    @functools.partial(
        pl.kernel,
        out_type=(
            jax.ShapeDtypeStruct((nc, B, F), jnp.float32),
            jax.ShapeDtypeStruct((NW, _L, B), jnp.float32),
        ),
        mesh=mesh,
        compiler_params=pltpu.CompilerParams(needs_layout_passes=False,
                                             use_tc_tiling_on_sc=False),
        scratch_types=[
            pltpu.VMEM((N,), jnp.int32),
            [pltpu.VMEM((NT, 128), jnp.int32)] * 2,
            [pltpu.VMEM((NP, NT, 8, 128), jnp.float32)] * 2,
            [pltpu.VMEM((C, F), jnp.float32)] * 2,
            [pltpu.VMEM((C,), jnp.int32)] * 2,
            pltpu.VMEM((1, 128), jnp.int32),
            pltpu.VMEM((NP, 1, 8, 128), jnp.float32),
            pltpu.VMEM((128, F), jnp.float32),
            pltpu.VMEM((128,), jnp.int32),
            pltpu.VMEM_SHARED((B, F), jnp.float32),
            pltpu.VMEM((_L, B), jnp.float32),
            [pltpu.SemaphoreType.DMA] * 2,
            [pltpu.SemaphoreType.DMA] * 2,
        ],
    )
    def edge_kernel(ei_hbm, attr_hbm, batch_hbm, sum_hbm, cnt_hbm,
                    batch_v, row_v, attr_v, rows_v, seg_v,
                    trow_v, tattr_v, trows_v, tseg_v, acc_s, cnt_v,
                    dsem, ssem):
        cid = lax.axis_index("c")
        sid = lax.axis_index("s")
        wid = sid * nc + cid

        zf = jnp.zeros((_L,), jnp.float32)
        zi = jnp.zeros((_L,), jnp.int32)

        def zrows(i, c):
            rows_v[0][i, :] = zf
            rows_v[1][i, :] = zf
            return c

        lax.fori_loop(0, C, zrows, 0)

        def zseg(i, c):
            seg_v[0][pl.ds(i * _L, _L)] = zi
            seg_v[1][pl.ds(i * _L, _L)] = zi
            return c

        lax.fori_loop(0, C // _L, zseg, 0)

        @pl.when(sid == 0)
        def _():
            pltpu.sync_copy(rows_v[0].at[pl.ds(0, B)], acc_s)

        def zcnt(i, c):
            cnt_v[i // (B // _L), pl.ds((i % (B // _L)) * _L, _L)] = zf
            return c

        lax.fori_loop(0, _L * (B // _L), zcnt, 0)

        lanes = lax.iota(jnp.int32, _L)
        ones = jnp.ones((_L,), jnp.float32)
        pidx = lanes // 8
        sidx = lanes % 8
        tile0 = wid * MAIN

        def start_dma(k, b):
            t = tile0 + k * NT
            pltpu.async_copy(ei_hbm.at[pl.ds(t, NT), 0, :], row_v[b], dsem[b])
            pltpu.async_copy(attr_hbm.at[:, pl.ds(t, NT)], attr_v[b], dsem[b])

        def wait_dma(b):
            pltpu.make_async_copy(ei_hbm.at[pl.ds(0, NT), 0, :], row_v[b],
                                  dsem[b]).wait()
            pltpu.make_async_copy(attr_hbm.at[:, pl.ds(0, NT)], attr_v[b],
                                  dsem[b]).wait()

        def wait_stream(b):
            pltpu.make_async_copy(rows_v[b], acc_s.at[seg_v[b]],
                                  ssem[b]).wait()

        def do_tiles(row_r, attr_r, rows_r, seg_r, nt):
            def tile_body(tt, c):
                tvec = jnp.full((_L,), tt, jnp.int32)

                def seg_body(jj, cc):
                    rv = row_r[tt, pl.ds(jj * _L, _L)]
                    seg = plsc.load_gather(batch_v, [rv])
                    seg_r[pl.ds(tt * 128 + jj * _L, _L)] = seg
                    plsc.addupdate_scatter(cnt_v, [lanes, seg], ones)
                    return cc

                lax.fori_loop(0, 128 // _L, seg_body, 0)

                def tr_body(j, cc):
                    for u in range(8):
                        l = j * 8 + u
                        vals = plsc.load_gather(
                            attr_r, [pidx, tvec, sidx,
                                     jnp.full((_L,), l, jnp.int32)])
                        rows_r[tt * 128 + l, :] = vals
                    return cc

                lax.fori_loop(0, 16, tr_body, 0)
                return c

            lax.fori_loop(0, nt, tile_body, 0)

        start_dma(0, 0)
        start_dma(1, 1)
        pltpu.sync_copy(batch_hbm, batch_v)
        plsc.subcore_barrier()
        for b in range(2):
            pltpu.async_copy(rows_v[b], acc_s.at[seg_v[b]], ssem[b], add=True)

        def pair_body(p, c):
            for b in range(2):
                k = 2 * p + b
                wait_dma(b)
                wait_stream(b)
                do_tiles(row_v[b], attr_v[b], rows_v[b], seg_v[b], NT)
                pltpu.async_copy(rows_v[b], acc_s.at[seg_v[b]], ssem[b],
                                 add=True)
                start_dma(k + 2, b)
            return c

        lax.fori_loop(0, KW // 2, pair_body, 0)
        for b in range(2):
            wait_stream(b)
            wait_dma(b)

        @pl.when(wid < TAIL)
        def _():
            t = NW * MAIN + wid
            pltpu.sync_copy(ei_hbm.at[pl.ds(t, 1), 0, :], trow_v)
            pltpu.sync_copy(attr_hbm.at[:, pl.ds(t, 1)], tattr_v)
            do_tiles(trow_v, tattr_v, trows_v, tseg_v, 1)
            pltpu.sync_copy(trows_v, acc_s.at[tseg_v], add=True)

        plsc.subcore_barrier()

        @pl.when(sid == 0)
        def _():
            pltpu.sync_copy(acc_s, sum_hbm.at[cid])

        pltpu.sync_copy(cnt_v, cnt_hbm.at[wid])

    return edge_kernel


def _make_node_kernel(N, D, B, NB):
    G = N // NB
    assert N % NB == 0

    def body(batch_ref, x_ref, sum_ref, cnt_ref):
        @pl.when(pl.program_id(0) == 0)
        def _():
            sum_ref[...] = jnp.zeros_like(sum_ref)
            cnt_ref[...] = jnp.zeros_like(cnt_ref)

        b = batch_ref[0, 0, :]
        onehot = (b[None, :] == lax.broadcasted_iota(jnp.int32, (B, NB), 0)
                  ).astype(jnp.float32)
        sum_ref[...] += jnp.dot(onehot, x_ref[...],
                                preferred_element_type=jnp.float32)
        cnt_ref[...] += jnp.broadcast_to(
            jnp.sum(onehot, axis=1, keepdims=True), cnt_ref.shape)

    return pl.pallas_call(
        body,
        grid=(G,),
        in_specs=[
            pl.BlockSpec((1, 1, NB), lambda i: (i, 0, 0)),
            pl.BlockSpec((NB, D), lambda i: (i, 0)),
        ],
        out_specs=[
            pl.BlockSpec((B, D), lambda i: (0, 0)),
            pl.BlockSpec((B, D), lambda i: (0, 0)),
        ],
        out_shape=[
            jax.ShapeDtypeStruct((B, D), jnp.float32),
            jax.ShapeDtypeStruct((B, D), jnp.float32),
        ],
    )


def _make_mlp_kernel(B, D, F, U, H, O):
    def body(u_ref, nsum_ref, ncnt_ref, esum_ref, ecnt_ref,
             w1_ref, b1_ref, w2_ref, b2_ref, out_ref):
        esum = jnp.sum(esum_ref[...], axis=0)
        ecnt = jnp.sum(ecnt_ref[...], axis=(0, 1))
        emean = esum / jnp.maximum(ecnt, 1.0)[:, None]
        nmean = nsum_ref[...] / jnp.maximum(ncnt_ref[...], 1.0)
        h = (jnp.dot(u_ref[...], w1_ref[0:U, :],
                     preferred_element_type=jnp.float32)
             + jnp.dot(nmean, w1_ref[U:U + D, :],
                       preferred_element_type=jnp.float32)
             + jnp.dot(emean, w1_ref[U + D:U + D + F, :],
                       preferred_element_type=jnp.float32)
             + b1_ref[...])
        h = jnp.maximum(h, 0.0)
        out_ref[...] = jnp.dot(h, w2_ref[...],
                               preferred_element_type=jnp.float32) + b2_ref[...]

    return pl.pallas_call(
        body,
        out_shape=jax.ShapeDtypeStruct((B, O), jnp.float32),
    )


def kernel(x, edge_index, edge_attr, u, batch, W1, b1, W2, b2):
    N, D = x.shape
    E, F = edge_attr.shape
    B, U = u.shape
    H = W1.shape[1]
    O = W2.shape[1]
    ntile = E // 128
    ei_v = edge_index.reshape(2, ntile, 128).transpose(1, 0, 2)
    x4 = edge_attr.T.reshape(F // 8, 8, ntile, 128).transpose(0, 2, 1, 3)

    info = plsc.get_sparse_core_info()
    esum, ecnt = _make_edge_kernel(E, F, N, B, info.num_cores,
                                   info.num_subcores)(ei_v, x4, batch)

    NB = 2000
    nsum, ncnt = _make_node_kernel(N, D, B, NB)(
        batch.reshape(N // NB, 1, NB), x)

    return _make_mlp_kernel(B, D, F, U, H, O)(
        u, nsum, ncnt, esum, ecnt, W1,
        b1.reshape(1, H), W2, b2.reshape(1, O))


# speedup vs baseline: 66.7940x; 2.5955x over previous
import functools

import jax
import jax.numpy as jnp
from jax import lax
from jax.experimental import pallas as pl
from jax.experimental.pallas import tpu as pltpu
from jax.experimental.pallas import tpu_sc as plsc

_L = 16


def _make_edge_kernel(E, F, N, B, nc, ns):
    NW = nc * ns
    NP = F // 8
    NTILES = E // 128
    MAIN = NTILES // NW
    TAIL = NTILES % NW
    NT = 5
    C = NT * 128
    assert E % 128 == 0 and MAIN % NT == 0 and F % 8 == 0
    KW = MAIN // NT
    assert KW % 2 == 0

    mesh = plsc.VectorSubcoreMesh(core_axis_name="c", subcore_axis_name="s")

    @functools.partial(
        pl.kernel,
        out_type=(
            jax.ShapeDtypeStruct((nc, B, F), jnp.float32),
            jax.ShapeDtypeStruct((NW, _L, B), jnp.float32),
        ),
        mesh=mesh,
        compiler_params=pltpu.CompilerParams(needs_layout_passes=False,
                                             use_tc_tiling_on_sc=False),
        scratch_types=[
            pltpu.VMEM((N,), jnp.int32),
            [pltpu.VMEM((NT, 128), jnp.int32)] * 2,
            [pltpu.VMEM((NP * NT * 8, 128), jnp.float32)] * 2,
            [pltpu.VMEM((C, F), jnp.float32)] * 2,
            [pltpu.VMEM((C,), jnp.int32)] * 2,
            pltpu.VMEM((1, 128), jnp.int32),
            pltpu.VMEM((NP * 8, 128), jnp.float32),
            pltpu.VMEM((128, F), jnp.float32),
            pltpu.VMEM((128,), jnp.int32),
            pltpu.VMEM_SHARED((B, F), jnp.float32),
            pltpu.VMEM((_L, B), jnp.float32),
            [pltpu.SemaphoreType.DMA] * 2,
            [pltpu.SemaphoreType.DMA] * 2,
        ],
    )
    def edge_kernel(ei_hbm, attr_hbm, batch_hbm, sum_hbm, cnt_hbm,
                    batch_v, row_v, attr_v, rows_v, seg_v,
                    trow_v, tattr_v, trows_v, tseg_v, acc_s, cnt_v,
                    dsem, ssem):
        cid = lax.axis_index("c")
        sid = lax.axis_index("s")
        wid = sid * nc + cid

        zf = jnp.zeros((_L,), jnp.float32)
        zi = jnp.zeros((_L,), jnp.int32)

        def zrows(i, c):
            rows_v[0][i, :] = zf
            return c

        lax.fori_loop(0, B, zrows, 0)

        @pl.when(sid == 0)
        def _():
            pltpu.sync_copy(rows_v[0].at[pl.ds(0, B)], acc_s)

        def zcnt(i, c):
            cnt_v[i // (B // _L), pl.ds((i % (B // _L)) * _L, _L)] = zf
            return c

        lax.fori_loop(0, _L * (B // _L), zcnt, 0)

        lanes = lax.iota(jnp.int32, _L)
        ones = jnp.ones((_L,), jnp.float32)
        pidx = lanes // 8
        sidx = lanes % 8
        tile0 = wid * MAIN

        def start_dma(k, b):
            t = tile0 + k * NT
            pltpu.async_copy(ei_hbm.at[pl.ds(t, NT), 0, :], row_v[b], dsem[b])
            for p in range(NP):
                pltpu.async_copy(
                    attr_hbm.at[pl.ds(p * NTILES * 8 + t * 8, NT * 8)],
                    attr_v[b].at[pl.ds(p * NT * 8, NT * 8)], dsem[b])

        def wait_dma(b):
            pltpu.make_async_copy(ei_hbm.at[pl.ds(0, NT), 0, :], row_v[b],
                                  dsem[b]).wait()
            for p in range(NP):
                pltpu.make_async_copy(
                    attr_hbm.at[pl.ds(0, NT * 8)],
                    attr_v[b].at[pl.ds(p * NT * 8, NT * 8)], dsem[b]).wait()

        arange16 = lax.iota(jnp.int32, _L)

        def do_tiles(row_r, attr_r, rows_r, seg_r, nt):
            def group_body(g, c):
                tt = g >> 3
                l0 = (g & 7) * _L
                rv = row_r[tt, pl.ds(l0, _L)]
                seg = plsc.load_gather(batch_v, [rv])
                seg_r[pl.ds(g * _L, _L)] = seg
                plsc.addupdate_scatter(cnt_v, [lanes, seg], ones)
                ridx = arange16 + g * _L
                for f in range(F):
                    vals = attr_r[(f // 8) * nt * 8 + tt * 8 + (f % 8),
                                  pl.ds(l0, _L)]
                    plsc.store_scatter(
                        rows_r, [ridx, jnp.full((_L,), f, jnp.int32)], vals)
                return c

            lax.fori_loop(0, nt * 8, group_body, 0)

        start_dma(0, 0)
        start_dma(1, 1)
        pltpu.sync_copy(batch_hbm, batch_v)
        plsc.subcore_barrier()

        def pair_body(p, c):
            sdesc = []
            for b in range(2):
                k = 2 * p + b
                wait_dma(b)
                do_tiles(row_v[b], attr_v[b], rows_v[b], seg_v[b], NT)
                sdesc.append(pltpu.async_copy(rows_v[b],
                                              acc_s.at[seg_v[b]],
                                              ssem[b], add=True))
                start_dma(k + 2, b)
            for d in sdesc:
                d.wait()
            return c

        lax.fori_loop(0, KW // 2, pair_body, 0)
        for b in range(2):
            wait_dma(b)

        @pl.when(wid < TAIL)
        def _():
            t = NW * MAIN + wid
            pltpu.sync_copy(ei_hbm.at[pl.ds(t, 1), 0, :], trow_v)
            for p in range(NP):
                pltpu.sync_copy(attr_hbm.at[pl.ds(p * NTILES * 8 + t * 8, 8)],
                                tattr_v.at[pl.ds(p * 8, 8)])
            do_tiles(trow_v, tattr_v, trows_v, tseg_v, 1)
            pltpu.sync_copy(trows_v, acc_s.at[tseg_v], add=True)

        plsc.subcore_barrier()

        @pl.when(sid == 0)
        def _():
            pltpu.sync_copy(acc_s, sum_hbm.at[cid])

        pltpu.sync_copy(cnt_v, cnt_hbm.at[wid])

    return edge_kernel


def _make_node_kernel(N, D, B, NB):
    G = N // NB
    assert N % NB == 0

    def body(batch_ref, x_ref, sum_ref, cnt_ref):
        @pl.when(pl.program_id(0) == 0)
        def _():
            sum_ref[...] = jnp.zeros_like(sum_ref)
            cnt_ref[...] = jnp.zeros_like(cnt_ref)

        b = batch_ref[0, 0, :]
        onehot = (b[None, :] == lax.broadcasted_iota(jnp.int32, (B, NB), 0)
                  ).astype(jnp.float32)
        sum_ref[...] += jnp.dot(onehot, x_ref[...],
                                preferred_element_type=jnp.float32)
        cnt_ref[...] += jnp.broadcast_to(
            jnp.sum(onehot, axis=1, keepdims=True), cnt_ref.shape)

    return pl.pallas_call(
        body,
        grid=(G,),
        in_specs=[
            pl.BlockSpec((1, 1, NB), lambda i: (i, 0, 0)),
            pl.BlockSpec((NB, D), lambda i: (i, 0)),
        ],
        out_specs=[
            pl.BlockSpec((B, D), lambda i: (0, 0)),
            pl.BlockSpec((B, D), lambda i: (0, 0)),
        ],
        out_shape=[
            jax.ShapeDtypeStruct((B, D), jnp.float32),
            jax.ShapeDtypeStruct((B, D), jnp.float32),
        ],
    )


def _make_mlp_kernel(B, D, F, U, H, O):
    def body(u_ref, nsum_ref, ncnt_ref, esum_ref, ecnt_ref,
             w1_ref, b1_ref, w2_ref, b2_ref, out_ref):
        esum = jnp.sum(esum_ref[...], axis=0)
        ecnt = jnp.sum(ecnt_ref[...], axis=(0, 1))
        emean = esum / jnp.maximum(ecnt, 1.0)[:, None]
        nmean = nsum_ref[...] / jnp.maximum(ncnt_ref[...], 1.0)
        h = (jnp.dot(u_ref[...], w1_ref[0:U, :],
                     preferred_element_type=jnp.float32)
             + jnp.dot(nmean, w1_ref[U:U + D, :],
                       preferred_element_type=jnp.float32)
             + jnp.dot(emean, w1_ref[U + D:U + D + F, :],
                       preferred_element_type=jnp.float32)
             + b1_ref[...])
        h = jnp.maximum(h, 0.0)
        out_ref[...] = jnp.dot(h, w2_ref[...],
                               preferred_element_type=jnp.float32) + b2_ref[...]

    return pl.pallas_call(
        body,
        out_shape=jax.ShapeDtypeStruct((B, O), jnp.float32),
    )


def kernel(x, edge_index, edge_attr, u, batch, W1, b1, W2, b2):
    N, D = x.shape
    E, F = edge_attr.shape
    B, U = u.shape
    H = W1.shape[1]
    O = W2.shape[1]
    ntile = E // 128
    ei_v = edge_index.reshape(2, ntile, 128).transpose(1, 0, 2)
    x2d = edge_attr.T.reshape(F // 8, 8, ntile, 128).transpose(
        0, 2, 1, 3).reshape(F // 8 * ntile * 8, 128)

    info = plsc.get_sparse_core_info()
    esum, ecnt = _make_edge_kernel(E, F, N, B, info.num_cores,
                                   info.num_subcores)(ei_v, x2d, batch)

    NB = 2000
    nsum, ncnt = _make_node_kernel(N, D, B, NB)(
        batch.reshape(N // NB, 1, NB), x)

    return _make_mlp_kernel(B, D, F, U, H, O)(
        u, nsum, ncnt, esum, ecnt, W1,
        b1.reshape(1, H), W2, b2.reshape(1, O))
